# baseline (speedup 1.0000x reference)
import jax
import jax.numpy as jnp
from jax.experimental import pallas as pl
from jax.experimental.pallas import tpu as pltpu

XD = 3
NM = 320
NPRO = 1280
NN = 1600
HID = 64
NLAYERS = 4
TI = 16
TJM = 64
TJP = 160
NI = NN // TI
NN_PAD = NN + TJP
INV_NORM = 0.01
F32 = jnp.float32


def _dot(a, b):
    return jax.lax.dot_general(a, b, (((1,), (0,)), ((), ())),
                               preferred_element_type=F32)


def _net_kernel(m_lo, m_trip, p_lo, p_trip,
                xm, xp, fm, fp, gid, RM, RP,
                em1w, em1b, em2w, em2b,
                ep1w, ep1b, ep2w, ep2b,
                wx, wh, bin_,
                A, B, c0, d1, d2, W2, b2,
                Nh, Na, nb1, N2, nb2,
                wov, bov, woh, boh,
                dm1w, dm1b, dm2w, dm2b,
                dp1w, dp1b, dp2w, dp2b,
                vm, vp, om, op,
                h_ref, p_ref, q_ref):
    silu = jax.nn.silu

    hm = _dot(silu(_dot(fm[:], em1w[:]) + em1b[:]), em2w[:]) + em2b[:]
    hp = _dot(silu(_dot(fp[:], ep1w[:]) + ep1b[:]), ep2w[:]) + ep2b[:]
    h_ref[0:NM, :] = _dot(xm[:], wx[:]) + _dot(hm, wh[:]) + bin_[:]
    h_ref[NM:NN, :] = _dot(xp[:], wx[:]) + _dot(hp, wh[:]) + bin_[:]
    h_ref[NN:NN_PAD, :] = jnp.zeros((NN_PAD - NN, HID), F32)

    for l in range(NLAYERS):
        a_l = A[l]
        b_l = B[l]
        d1_l = d1[l]
        d2_l = d2[l]
        w2_l = W2[l]
        b2_l = b2[l]
        nh_l = Nh[l]
        na_l = Na[l]
        nb1_l = nb1[l]
        n2_l = N2[l]
        nb2_l = nb2[l]
        p_ref[:] = _dot(h_ref[:], a_l) + c0[l]
        q_ref[:] = _dot(h_ref[:], b_l)

        def i_body(it, carry):
            i0 = it * TI
            u = p_ref[pl.ds(i0, TI), :]
            gi3 = gid[pl.ds(i0, TI), :][:, :, None]
            ii = jax.lax.broadcasted_iota(jnp.int32, (TI, 1, 1), 0) + i0
            mol_i3 = ii < NM
            u3 = u[:, None, :]
            zero = jnp.zeros_like(d1_l)[None]
            u3m = u3 + jnp.where(mol_i3, d1_l[None], zero)
            u3p = u3 + jnp.where(mol_i3, zero, d2_l[None])

            def mol_j_body(k, acc):
                j0 = m_lo[it] + k * TJM
                v3 = q_ref[pl.ds(j0, TJM), :][None, :, :]
                gj3 = gid[pl.ds(j0, TJM), :][None, :, :]
                jj = jax.lax.broadcasted_iota(jnp.int32, (1, TJM, 1), 1) + j0
                s = silu(u3m + v3)
                m = silu(_dot(s.reshape(TI * TJM, HID), w2_l) + b2_l)
                msk = ((gi3 == gj3) & (jj < NM)).astype(F32)
                m2 = (m.reshape(TI, TJM, HID) * msk).reshape(TI * TJM, HID)
                return acc + _dot(RM[:], m2)

            def pro_j_body(k, acc):
                j0 = p_lo[it] + k * TJP
                v3 = q_ref[pl.ds(j0, TJP), :][None, :, :]
                gj3 = gid[pl.ds(j0, TJP), :][None, :, :]
                s = silu(u3p + v3)
                m = silu(_dot(s.reshape(TI * TJP, HID), w2_l) + b2_l)
                msk = (gi3 == gj3).astype(F32)
                m2 = (m.reshape(TI, TJP, HID) * msk).reshape(TI * TJP, HID)
                return acc + _dot(RP[:], m2)

            agg = jax.lax.fori_loop(0, m_trip[it], mol_j_body,
                                    jnp.zeros((TI, HID), F32))
            agg = jax.lax.fori_loop(0, p_trip[it], pro_j_body, agg)
            hrow = h_ref[pl.ds(i0, TI), :]
            hn = _dot(silu(_dot(hrow, nh_l) + _dot(agg, na_l) + nb1_l),
                      n2_l) + nb2_l
            h_ref[pl.ds(i0, TI), :] = hrow + hn
            return carry

        jax.lax.fori_loop(0, NI, i_body, 0)

    hfin_m = h_ref[0:NM, :]
    hfin_p = h_ref[NM:NN, :]
    vm[:] = _dot(hfin_m, wov[:]) + bov[:]
    vp[:] = _dot(hfin_p, wov[:]) + bov[:]
    hm2 = _dot(hfin_m, woh[:]) + boh[:]
    hp2 = _dot(hfin_p, woh[:]) + boh[:]
    om[:] = _dot(silu(_dot(hm2, dm1w[:]) + dm1b[:]), dm2w[:]) + dm2b[:]
    op[:] = _dot(silu(_dot(hp2, dp1w[:]) + dp1b[:]), dp2w[:]) + dp2b[:]


def kernel(z_t_mol, z_t_pro, t, molecule_idx, protein_pocket_idx, params):
    p = params
    xm = z_t_mol[:, :XD]
    fm = z_t_mol[:, XD:]
    xp = z_t_pro[:, :XD]
    fp = z_t_pro[:, XD:]
    mol_idx = molecule_idx.astype(jnp.int32)
    pro_idx = protein_pocket_idx.astype(jnp.int32)
    gid_flat = jnp.concatenate([mol_idx, pro_idx])
    gid = jnp.pad(gid_flat, (0, NN_PAD - NN), constant_values=-1)[:, None]

    i0s = jnp.arange(NI, dtype=jnp.int32) * TI
    glo = gid_flat[i0s]
    ghi = gid_flat[i0s + TI - 1]
    m_lo_raw = jnp.searchsorted(mol_idx, glo, side="left").astype(jnp.int32)
    m_hi = jnp.searchsorted(mol_idx, ghi, side="right").astype(jnp.int32)
    p_lo_raw = NM + jnp.searchsorted(pro_idx, glo, side="left").astype(
        jnp.int32)
    p_hi = NM + jnp.searchsorted(pro_idx, ghi, side="right").astype(jnp.int32)
    m_lo = (m_lo_raw // 8) * 8
    p_lo = (p_lo_raw // 8) * 8
    m_trip = jnp.maximum(0, (m_hi - m_lo + TJM - 1) // TJM)
    p_trip = jnp.maximum(0, (p_hi - p_lo + TJP - 1) // TJP)

    def wt(q):
        return q["w"].T

    def bt(q):
        return q["b"][None, :]

    gcl = p["gcl"]
    emb = p["edge_embedding"]
    A = jnp.stack([g["edge_mlp"][0]["w"][:, :HID].T for g in gcl])
    B = jnp.stack([g["edge_mlp"][0]["w"][:, HID:2 * HID].T for g in gcl])
    cT = jnp.stack([emb @ g["edge_mlp"][0]["w"][:, 2 * HID:].T
                    + g["edge_mlp"][0]["b"][None, :] for g in gcl])
    c0 = cT[:, 0:1, :]
    d1 = cT[:, 1:2, :] - c0
    d2 = cT[:, 2:3, :] - c0
    W2 = jnp.stack([wt(g["edge_mlp"][1]) for g in gcl])
    b2 = jnp.stack([bt(g["edge_mlp"][1]) for g in gcl])
    Nh = jnp.stack([g["node_mlp"][0]["w"][:, :HID].T for g in gcl])
    Na = jnp.stack([g["node_mlp"][0]["w"][:, HID:].T for g in gcl]) * INV_NORM
    nb1 = jnp.stack([bt(g["node_mlp"][0]) for g in gcl])
    N2 = jnp.stack([wt(g["node_mlp"][1]) for g in gcl])
    nb2 = jnp.stack([bt(g["node_mlp"][1]) for g in gcl])

    gi_w = p["gnn_in"]["w"]
    go_w = p["gnn_out"]["w"]
    go_b = p["gnn_out"]["b"]

    RM = jnp.repeat(jnp.eye(TI, dtype=F32), TJM, axis=1)
    RP = jnp.repeat(jnp.eye(TI, dtype=F32), TJP, axis=1)

    smem_ins = [m_lo, m_trip, p_lo, p_trip]
    vmem_ins = [
        xm, xp, fm, fp, gid, RM, RP,
        wt(p["atom_enc"][0]), bt(p["atom_enc"][0]),
        wt(p["atom_enc"][1]), bt(p["atom_enc"][1]),
        wt(p["res_enc"][0]), bt(p["res_enc"][0]),
        wt(p["res_enc"][1]), bt(p["res_enc"][1]),
        gi_w[:, :XD].T, gi_w[:, XD:].T, p["gnn_in"]["b"][None, :],
        A, B, c0, d1, d2, W2, b2,
        Nh, Na, nb1, N2, nb2,
        go_w[:XD].T, go_b[None, :XD], go_w[XD:].T, go_b[None, XD:],
        wt(p["atom_dec"][0]), bt(p["atom_dec"][0]),
        wt(p["atom_dec"][1]), bt(p["atom_dec"][1]),
        wt(p["res_dec"][0]), bt(p["res_dec"][0]),
        wt(p["res_dec"][1]), bt(p["res_dec"][1]),
    ]
    in_specs = ([pl.BlockSpec(memory_space=pltpu.SMEM)] * len(smem_ins)
                + [pl.BlockSpec(memory_space=pltpu.VMEM)] * len(vmem_ins))

    out_shape = [
        jax.ShapeDtypeStruct((NM, XD), F32),
        jax.ShapeDtypeStruct((NPRO, XD), F32),
        jax.ShapeDtypeStruct((NM, 16), F32),
        jax.ShapeDtypeStruct((NPRO, 20), F32),
    ]
    vm, vp, om, op = pl.pallas_call(
        _net_kernel,
        out_shape=out_shape,
        in_specs=in_specs,
        scratch_shapes=[pltpu.VMEM((NN_PAD, HID), F32)] * 3,
    )(*smem_ins, *vmem_ins)
    eps_mol = jnp.concatenate([vm, om], axis=1)
    eps_pro = jnp.concatenate([vp, op], axis=1)
    return eps_mol, eps_pro


# speedup vs baseline: 1.1174x; 1.1174x over previous
import jax
import jax.numpy as jnp
from jax.experimental import pallas as pl
from jax.experimental.pallas import tpu as pltpu

XD = 3
NM = 320
NPRO = 1280
NN = 1600
HID = 64
NLAYERS = 4
TI = 16
TJM = 64
TJP = 160
NI = NN // TI
NN_PAD = NN + TJP
INV_NORM = 0.01
F32 = jnp.float32


def _dot(a, b):
    return jax.lax.dot_general(a, b, (((1,), (0,)), ((), ())),
                               preferred_element_type=F32)


def _net_kernel(m_lo, m_trip, p_lo, p_trip,
                xm, xp, fm, fp, gid, RM, RP,
                em1w, em1b, em2w, em2b,
                ep1w, ep1b, ep2w, ep2b,
                wx, wh, bin_,
                A, B, c0, d1, d2, W2, b2,
                Nh, Na, nb1, N2, nb2,
                wov, bov, woh, boh,
                dm1w, dm1b, dm2w, dm2b,
                dp1w, dp1b, dp2w, dp2b,
                vm, vp, om, op,
                h_ref, p_ref, q_ref):
    silu = jax.nn.silu

    hm = _dot(silu(_dot(fm[:], em1w[:]) + em1b[:]), em2w[:]) + em2b[:]
    hp = _dot(silu(_dot(fp[:], ep1w[:]) + ep1b[:]), ep2w[:]) + ep2b[:]
    h_ref[0:NM, :] = _dot(xm[:], wx[:]) + _dot(hm, wh[:]) + bin_[:]
    h_ref[NM:NN, :] = _dot(xp[:], wx[:]) + _dot(hp, wh[:]) + bin_[:]
    h_ref[NN:NN_PAD, :] = jnp.zeros((NN_PAD - NN, HID), F32)

    for l in range(NLAYERS):
        a_l = A[l]
        b_l = B[l]
        d1_l = d1[l]
        d2_l = d2[l]
        w2_l = W2[l]
        b2_l = b2[l]
        nh_l = Nh[l]
        na_l = Na[l]
        nb1_l = nb1[l]
        n2_l = N2[l]
        nb2_l = nb2[l]
        p_ref[:] = _dot(h_ref[:], a_l) + c0[l]
        q_ref[:] = _dot(h_ref[:], b_l)

        def i_body(it, carry):
            i0 = it * TI
            u = p_ref[pl.ds(i0, TI), :]
            gi3 = gid[pl.ds(i0, TI), :][:, :, None]
            ii = jax.lax.broadcasted_iota(jnp.int32, (TI, 1, 1), 0) + i0
            mol_i3 = ii < NM
            u3 = u[:, None, :]
            zero = jnp.zeros_like(d1_l)[None]
            u3m = u3 + jnp.where(mol_i3, d1_l[None], zero)
            u3p = u3 + jnp.where(mol_i3, zero, d2_l[None])

            def mol_j_body(k, acc):
                j0 = m_lo[it] + k * TJM
                v3 = q_ref[pl.ds(j0, TJM), :][None, :, :]
                gj3 = gid[pl.ds(j0, TJM), :][None, :, :]
                jj = jax.lax.broadcasted_iota(jnp.int32, (1, TJM, 1), 1) + j0
                s = silu(u3m + v3)
                m = silu(_dot(s.reshape(TI * TJM, HID), w2_l) + b2_l)
                msk = ((gi3 == gj3) & (jj < NM)).astype(F32)
                return acc + jnp.sum(m.reshape(TI, TJM, HID) * msk, axis=1)

            def pro_j_body(k, acc):
                j0 = p_lo[it] + k * TJP
                v3 = q_ref[pl.ds(j0, TJP), :][None, :, :]
                gj3 = gid[pl.ds(j0, TJP), :][None, :, :]
                s = silu(u3p + v3)
                m = silu(_dot(s.reshape(TI * TJP, HID), w2_l) + b2_l)
                msk = (gi3 == gj3).astype(F32)
                return acc + jnp.sum(m.reshape(TI, TJP, HID) * msk, axis=1)

            agg = jax.lax.fori_loop(0, m_trip[it], mol_j_body,
                                    jnp.zeros((TI, HID), F32))
            agg = jax.lax.fori_loop(0, p_trip[it], pro_j_body, agg)
            hrow = h_ref[pl.ds(i0, TI), :]
            hn = _dot(silu(_dot(hrow, nh_l) + _dot(agg, na_l) + nb1_l),
                      n2_l) + nb2_l
            h_ref[pl.ds(i0, TI), :] = hrow + hn
            return carry

        jax.lax.fori_loop(0, NI, i_body, 0)

    hfin_m = h_ref[0:NM, :]
    hfin_p = h_ref[NM:NN, :]
    vm[:] = _dot(hfin_m, wov[:]) + bov[:]
    vp[:] = _dot(hfin_p, wov[:]) + bov[:]
    hm2 = _dot(hfin_m, woh[:]) + boh[:]
    hp2 = _dot(hfin_p, woh[:]) + boh[:]
    om[:] = _dot(silu(_dot(hm2, dm1w[:]) + dm1b[:]), dm2w[:]) + dm2b[:]
    op[:] = _dot(silu(_dot(hp2, dp1w[:]) + dp1b[:]), dp2w[:]) + dp2b[:]


def kernel(z_t_mol, z_t_pro, t, molecule_idx, protein_pocket_idx, params):
    p = params
    xm = z_t_mol[:, :XD]
    fm = z_t_mol[:, XD:]
    xp = z_t_pro[:, :XD]
    fp = z_t_pro[:, XD:]
    mol_idx = molecule_idx.astype(jnp.int32)
    pro_idx = protein_pocket_idx.astype(jnp.int32)
    gid_flat = jnp.concatenate([mol_idx, pro_idx])
    gid = jnp.pad(gid_flat, (0, NN_PAD - NN), constant_values=-1)[:, None]

    i0s = jnp.arange(NI, dtype=jnp.int32) * TI
    glo = gid_flat[i0s]
    ghi = gid_flat[i0s + TI - 1]
    m_lo_raw = jnp.searchsorted(mol_idx, glo, side="left").astype(jnp.int32)
    m_hi = jnp.searchsorted(mol_idx, ghi, side="right").astype(jnp.int32)
    p_lo_raw = NM + jnp.searchsorted(pro_idx, glo, side="left").astype(
        jnp.int32)
    p_hi = NM + jnp.searchsorted(pro_idx, ghi, side="right").astype(jnp.int32)
    m_lo = (m_lo_raw // 8) * 8
    p_lo = (p_lo_raw // 8) * 8
    m_trip = jnp.maximum(0, (m_hi - m_lo + TJM - 1) // TJM)
    p_trip = jnp.maximum(0, (p_hi - p_lo + TJP - 1) // TJP)

    def wt(q):
        return q["w"].T

    def bt(q):
        return q["b"][None, :]

    gcl = p["gcl"]
    emb = p["edge_embedding"]
    A = jnp.stack([g["edge_mlp"][0]["w"][:, :HID].T for g in gcl])
    B = jnp.stack([g["edge_mlp"][0]["w"][:, HID:2 * HID].T for g in gcl])
    cT = jnp.stack([emb @ g["edge_mlp"][0]["w"][:, 2 * HID:].T
                    + g["edge_mlp"][0]["b"][None, :] for g in gcl])
    c0 = cT[:, 0:1, :]
    d1 = cT[:, 1:2, :] - c0
    d2 = cT[:, 2:3, :] - c0
    W2 = jnp.stack([wt(g["edge_mlp"][1]) for g in gcl])
    b2 = jnp.stack([bt(g["edge_mlp"][1]) for g in gcl])
    Nh = jnp.stack([g["node_mlp"][0]["w"][:, :HID].T for g in gcl])
    Na = jnp.stack([g["node_mlp"][0]["w"][:, HID:].T for g in gcl]) * INV_NORM
    nb1 = jnp.stack([bt(g["node_mlp"][0]) for g in gcl])
    N2 = jnp.stack([wt(g["node_mlp"][1]) for g in gcl])
    nb2 = jnp.stack([bt(g["node_mlp"][1]) for g in gcl])

    gi_w = p["gnn_in"]["w"]
    go_w = p["gnn_out"]["w"]
    go_b = p["gnn_out"]["b"]

    RM = jnp.repeat(jnp.eye(TI, dtype=F32), TJM, axis=1)
    RP = jnp.repeat(jnp.eye(TI, dtype=F32), TJP, axis=1)

    smem_ins = [m_lo, m_trip, p_lo, p_trip]
    vmem_ins = [
        xm, xp, fm, fp, gid, RM, RP,
        wt(p["atom_enc"][0]), bt(p["atom_enc"][0]),
        wt(p["atom_enc"][1]), bt(p["atom_enc"][1]),
        wt(p["res_enc"][0]), bt(p["res_enc"][0]),
        wt(p["res_enc"][1]), bt(p["res_enc"][1]),
        gi_w[:, :XD].T, gi_w[:, XD:].T, p["gnn_in"]["b"][None, :],
        A, B, c0, d1, d2, W2, b2,
        Nh, Na, nb1, N2, nb2,
        go_w[:XD].T, go_b[None, :XD], go_w[XD:].T, go_b[None, XD:],
        wt(p["atom_dec"][0]), bt(p["atom_dec"][0]),
        wt(p["atom_dec"][1]), bt(p["atom_dec"][1]),
        wt(p["res_dec"][0]), bt(p["res_dec"][0]),
        wt(p["res_dec"][1]), bt(p["res_dec"][1]),
    ]
    in_specs = ([pl.BlockSpec(memory_space=pltpu.SMEM)] * len(smem_ins)
                + [pl.BlockSpec(memory_space=pltpu.VMEM)] * len(vmem_ins))

    out_shape = [
        jax.ShapeDtypeStruct((NM, XD), F32),
        jax.ShapeDtypeStruct((NPRO, XD), F32),
        jax.ShapeDtypeStruct((NM, 16), F32),
        jax.ShapeDtypeStruct((NPRO, 20), F32),
    ]
    vm, vp, om, op = pl.pallas_call(
        _net_kernel,
        out_shape=out_shape,
        in_specs=in_specs,
        scratch_shapes=[pltpu.VMEM((NN_PAD, HID), F32)] * 3,
    )(*smem_ins, *vmem_ins)
    eps_mol = jnp.concatenate([vm, om], axis=1)
    eps_pro = jnp.concatenate([vp, op], axis=1)
    return eps_mol, eps_pro


# speedup vs baseline: 1.1971x; 1.0713x over previous
import jax
import jax.numpy as jnp
from jax.experimental import pallas as pl
from jax.experimental.pallas import tpu as pltpu

XD = 3
NM = 320
NPRO = 1280
NN = 1600
HID = 64
NLAYERS = 4
TI = 16
TJM = 64
TJP = 128
NI = NN // TI
NN_PAD = NN + TJP
INV_NORM = 0.01
F32 = jnp.float32


def _dot(a, b):
    return jax.lax.dot_general(a, b, (((1,), (0,)), ((), ())),
                               preferred_element_type=F32)


def _net_kernel(m_lo, m_trip, p_lo, p_trip,
                xm, xp, fm, fp, gid, RM, RP,
                em1w, em1b, em2w, em2b,
                ep1w, ep1b, ep2w, ep2b,
                wx, wh, bin_,
                A, B, c0, d1, d2, W2, b2,
                Nh, Na, nb1, N2, nb2,
                wov, bov, woh, boh,
                dm1w, dm1b, dm2w, dm2b,
                dp1w, dp1b, dp2w, dp2b,
                vm, vp, om, op,
                h_ref, p_ref, q_ref):
    silu = jax.nn.silu

    hm = _dot(silu(_dot(fm[:], em1w[:]) + em1b[:]), em2w[:]) + em2b[:]
    hp = _dot(silu(_dot(fp[:], ep1w[:]) + ep1b[:]), ep2w[:]) + ep2b[:]
    h_ref[0:NM, :] = _dot(xm[:], wx[:]) + _dot(hm, wh[:]) + bin_[:]
    h_ref[NM:NN, :] = _dot(xp[:], wx[:]) + _dot(hp, wh[:]) + bin_[:]
    h_ref[NN:NN_PAD, :] = jnp.zeros((NN_PAD - NN, HID), F32)

    for l in range(NLAYERS):
        a_l = A[l]
        b_l = B[l]
        d1_l = d1[l]
        d2_l = d2[l]
        w2_l = W2[l]
        b2_l = b2[l]
        nh_l = Nh[l]
        na_l = Na[l]
        nb1_l = nb1[l]
        n2_l = N2[l]
        nb2_l = nb2[l]
        p_ref[:] = _dot(h_ref[:], a_l) + c0[l]
        q_ref[:] = _dot(h_ref[:], b_l)

        def i_body(it, carry):
            i0 = it * TI
            u = p_ref[pl.ds(i0, TI), :]
            gi3 = gid[pl.ds(i0, TI), :][:, :, None]
            ii = jax.lax.broadcasted_iota(jnp.int32, (TI, 1, 1), 0) + i0
            mol_i3 = ii < NM
            u3 = u[:, None, :]
            zero = jnp.zeros_like(d1_l)[None]
            u3m = u3 + jnp.where(mol_i3, d1_l[None], zero)
            u3p = u3 + jnp.where(mol_i3, zero, d2_l[None])

            def mol_j_body(k, acc):
                j0 = m_lo[it] + k * TJM
                v3 = q_ref[pl.ds(j0, TJM), :][None, :, :]
                gj3 = gid[pl.ds(j0, TJM), :][None, :, :]
                jj = jax.lax.broadcasted_iota(jnp.int32, (1, TJM, 1), 1) + j0
                s = silu(u3m + v3)
                m = silu(_dot(s.reshape(TI * TJM, HID), w2_l) + b2_l)
                msk = ((gi3 == gj3) & (jj < NM)).astype(F32)
                return acc + jnp.sum(m.reshape(TI, TJM, HID) * msk, axis=1)

            def pro_j_body(k, acc):
                j0 = p_lo[it] + k * TJP
                v3 = q_ref[pl.ds(j0, TJP), :][None, :, :]
                gj3 = gid[pl.ds(j0, TJP), :][None, :, :]
                s = silu(u3p + v3)
                m = silu(_dot(s.reshape(TI * TJP, HID), w2_l) + b2_l)
                msk = (gi3 == gj3).astype(F32)
                return acc + jnp.sum(m.reshape(TI, TJP, HID) * msk, axis=1)

            agg = jax.lax.fori_loop(0, m_trip[it], mol_j_body,
                                    jnp.zeros((TI, HID), F32))
            agg = jax.lax.fori_loop(0, p_trip[it], pro_j_body, agg)
            hrow = h_ref[pl.ds(i0, TI), :]
            hn = _dot(silu(_dot(hrow, nh_l) + _dot(agg, na_l) + nb1_l),
                      n2_l) + nb2_l
            h_ref[pl.ds(i0, TI), :] = hrow + hn
            return carry

        jax.lax.fori_loop(0, NI, i_body, 0)

    hfin_m = h_ref[0:NM, :]
    hfin_p = h_ref[NM:NN, :]
    vm[:] = _dot(hfin_m, wov[:]) + bov[:]
    vp[:] = _dot(hfin_p, wov[:]) + bov[:]
    hm2 = _dot(hfin_m, woh[:]) + boh[:]
    hp2 = _dot(hfin_p, woh[:]) + boh[:]
    om[:] = _dot(silu(_dot(hm2, dm1w[:]) + dm1b[:]), dm2w[:]) + dm2b[:]
    op[:] = _dot(silu(_dot(hp2, dp1w[:]) + dp1b[:]), dp2w[:]) + dp2b[:]


def kernel(z_t_mol, z_t_pro, t, molecule_idx, protein_pocket_idx, params):
    p = params
    xm = z_t_mol[:, :XD]
    fm = z_t_mol[:, XD:]
    xp = z_t_pro[:, :XD]
    fp = z_t_pro[:, XD:]
    mol_idx = molecule_idx.astype(jnp.int32)
    pro_idx = protein_pocket_idx.astype(jnp.int32)
    gid_flat = jnp.concatenate([mol_idx, pro_idx])
    gid = jnp.pad(gid_flat, (0, NN_PAD - NN), constant_values=-1)[:, None]

    i0s = jnp.arange(NI, dtype=jnp.int32) * TI
    glo = gid_flat[i0s]
    ghi = gid_flat[i0s + TI - 1]
    m_lo_raw = jnp.searchsorted(mol_idx, glo, side="left").astype(jnp.int32)
    m_hi = jnp.searchsorted(mol_idx, ghi, side="right").astype(jnp.int32)
    p_lo_raw = NM + jnp.searchsorted(pro_idx, glo, side="left").astype(
        jnp.int32)
    p_hi = NM + jnp.searchsorted(pro_idx, ghi, side="right").astype(jnp.int32)
    m_lo = (m_lo_raw // 8) * 8
    p_lo = (p_lo_raw // 8) * 8
    m_trip = jnp.maximum(0, (m_hi - m_lo + TJM - 1) // TJM)
    p_trip = jnp.maximum(0, (p_hi - p_lo + TJP - 1) // TJP)

    def wt(q):
        return q["w"].T

    def bt(q):
        return q["b"][None, :]

    gcl = p["gcl"]
    emb = p["edge_embedding"]
    A = jnp.stack([g["edge_mlp"][0]["w"][:, :HID].T for g in gcl])
    B = jnp.stack([g["edge_mlp"][0]["w"][:, HID:2 * HID].T for g in gcl])
    cT = jnp.stack([emb @ g["edge_mlp"][0]["w"][:, 2 * HID:].T
                    + g["edge_mlp"][0]["b"][None, :] for g in gcl])
    c0 = cT[:, 0:1, :]
    d1 = cT[:, 1:2, :] - c0
    d2 = cT[:, 2:3, :] - c0
    W2 = jnp.stack([wt(g["edge_mlp"][1]) for g in gcl])
    b2 = jnp.stack([bt(g["edge_mlp"][1]) for g in gcl])
    Nh = jnp.stack([g["node_mlp"][0]["w"][:, :HID].T for g in gcl])
    Na = jnp.stack([g["node_mlp"][0]["w"][:, HID:].T for g in gcl]) * INV_NORM
    nb1 = jnp.stack([bt(g["node_mlp"][0]) for g in gcl])
    N2 = jnp.stack([wt(g["node_mlp"][1]) for g in gcl])
    nb2 = jnp.stack([bt(g["node_mlp"][1]) for g in gcl])

    gi_w = p["gnn_in"]["w"]
    go_w = p["gnn_out"]["w"]
    go_b = p["gnn_out"]["b"]

    RM = jnp.repeat(jnp.eye(TI, dtype=F32), TJM, axis=1)
    RP = jnp.repeat(jnp.eye(TI, dtype=F32), TJP, axis=1)

    smem_ins = [m_lo, m_trip, p_lo, p_trip]
    vmem_ins = [
        xm, xp, fm, fp, gid, RM, RP,
        wt(p["atom_enc"][0]), bt(p["atom_enc"][0]),
        wt(p["atom_enc"][1]), bt(p["atom_enc"][1]),
        wt(p["res_enc"][0]), bt(p["res_enc"][0]),
        wt(p["res_enc"][1]), bt(p["res_enc"][1]),
        gi_w[:, :XD].T, gi_w[:, XD:].T, p["gnn_in"]["b"][None, :],
        A, B, c0, d1, d2, W2, b2,
        Nh, Na, nb1, N2, nb2,
        go_w[:XD].T, go_b[None, :XD], go_w[XD:].T, go_b[None, XD:],
        wt(p["atom_dec"][0]), bt(p["atom_dec"][0]),
        wt(p["atom_dec"][1]), bt(p["atom_dec"][1]),
        wt(p["res_dec"][0]), bt(p["res_dec"][0]),
        wt(p["res_dec"][1]), bt(p["res_dec"][1]),
    ]
    in_specs = ([pl.BlockSpec(memory_space=pltpu.SMEM)] * len(smem_ins)
                + [pl.BlockSpec(memory_space=pltpu.VMEM)] * len(vmem_ins))

    out_shape = [
        jax.ShapeDtypeStruct((NM, XD), F32),
        jax.ShapeDtypeStruct((NPRO, XD), F32),
        jax.ShapeDtypeStruct((NM, 16), F32),
        jax.ShapeDtypeStruct((NPRO, 20), F32),
    ]
    vm, vp, om, op = pl.pallas_call(
        _net_kernel,
        out_shape=out_shape,
        in_specs=in_specs,
        scratch_shapes=[pltpu.VMEM((NN_PAD, HID), F32)] * 3,
    )(*smem_ins, *vmem_ins)
    eps_mol = jnp.concatenate([vm, om], axis=1)
    eps_pro = jnp.concatenate([vp, op], axis=1)
    return eps_mol, eps_pro
